# baseline (speedup 1.0000x reference)
import math

import jax
import jax.numpy as jnp
from jax import lax
from jax.experimental import pallas as pl
from jax.experimental.pallas import tpu as pltpu
from jax.experimental.pallas import tpu_sc as plsc

NUM_CORES = 2
NUM_SUBCORES = 16
NUM_WORKERS = NUM_CORES * NUM_SUBCORES
LANES = 16
HALF = 256


def _emb_body(xt_hbm, lut_hbm, out_hbm, idx_v, rows_v, tile_v, g0, g1, s0,
              s1):
    h, nb = idx_v.shape
    d = rows_v.shape[-1]
    ta = d // 8
    scale = jnp.float32(math.sqrt(d))
    wid = lax.axis_index("s") * NUM_CORES + lax.axis_index("c")
    base = wid * nb
    gsem = (g0, g1)
    ssem = (s0, s1)

    row_iota = lax.iota(jnp.int32, LANES)

    def fire_gather(hh, hf, slot):
        for sub in range(HALF // 128):
            pltpu.async_copy(
                lut_hbm.at[idx_v.at[hh, pl.ds(hf * HALF + sub * 128, 128)]],
                rows_v.at[slot, pl.ds(sub * 128, 128)], gsem[slot])

    def drain_gather(slot):
        for sub in range(HALF // 128):
            pltpu.make_async_copy(
                lut_hbm.at[idx_v.at[0, pl.ds(0, 128)]],
                rows_v.at[slot, pl.ds(sub * 128, 128)], gsem[slot]).wait()

    def fire_store(hh, hf, slot):
        for a in range(ta):
            pltpu.async_copy(
                tile_v.at[slot, a],
                out_hbm.at[hh, a, pl.ds(wid * 4 + hf * 2, 2)], ssem[slot])

    def drain_store(slot):
        for a in range(ta):
            pltpu.make_async_copy(
                tile_v.at[slot, a], out_hbm.at[0, 0, pl.ds(0, 2)],
                ssem[slot]).wait()

    def compute(slot):
        rows2 = rows_v.at[slot]
        for jj in range(2):
            for c16 in range(HALF // 2 // LANES):
                c0 = c16 * LANES
                rowvec = row_iota + (jj * 128 + c0)

                def dbody(dd, _, rowvec=rowvec, jj=jj, c0=c0):
                    colvec = jnp.zeros((LANES,), jnp.int32) + dd
                    val = plsc.load_gather(rows2, [rowvec, colvec])
                    tile_v[slot, dd >> 3, jj, dd & 7, pl.ds(c0, LANES)] = (
                        val * scale)
                    return 0

                lax.fori_loop(0, d, dbody, 0, unroll=8)

    pltpu.sync_copy(xt_hbm.at[:, pl.ds(base, nb)], idx_v)

    fire_gather(0, 0, 0)

    def pair_body(t, _):
        for p in range(2):
            @pl.when(t >= 1)
            def _():
                drain_store(p)
            if p == 0:
                fire_gather(t, 1, 1)
            else:
                @pl.when(t < h - 1)
                def _():
                    fire_gather(t + 1, 0, 0)
            drain_gather(p)
            compute(p)
            fire_store(t, p, p)
        return 0

    lax.fori_loop(0, h, pair_body, 0)

    drain_store(0)
    drain_store(1)


def kernel(x, lut):
    b, h = x.shape
    v, d = lut.shape
    nb = b // NUM_WORKERS
    assert b % (NUM_WORKERS * HALF) == 0
    assert d % LANES == 0 and d % 8 == 0

    xt = x.astype(jnp.int32).T

    mesh = plsc.VectorSubcoreMesh(core_axis_name="c", subcore_axis_name="s")
    run = pl.kernel(
        _emb_body,
        out_type=jax.ShapeDtypeStruct((h, d // 8, b // 128, 8, 128),
                                      jnp.float32),
        mesh=mesh,
        scratch_types=[
            pltpu.VMEM((h, nb), jnp.int32),
            pltpu.VMEM((2, HALF, d), jnp.float32),
            pltpu.VMEM((2, d // 8, 2, 8, 128), jnp.float32),
            pltpu.SemaphoreType.DMA,
            pltpu.SemaphoreType.DMA,
            pltpu.SemaphoreType.DMA,
            pltpu.SemaphoreType.DMA,
        ],
        compiler_params=pltpu.CompilerParams(
            use_tc_tiling_on_sc=False, needs_layout_passes=False
        ),
    )
    buf = run(xt, lut)
    out = buf.transpose(2, 4, 0, 1, 3).reshape(b, h, d)
    return out


# speedup vs baseline: 1.9987x; 1.9987x over previous
import math

import jax
import jax.numpy as jnp
from jax import lax
from jax.experimental import pallas as pl
from jax.experimental.pallas import tpu as pltpu
from jax.experimental.pallas import tpu_sc as plsc

NUM_CORES = 2
NUM_SUBCORES = 16
NUM_WORKERS = NUM_CORES * NUM_SUBCORES
LANES = 16
NSLOT = 4


def _emb_body(x_hbm, lut_hbm, out_hbm, idx_v, rows_v, g0, g1, g2, g3, s0,
              s1, s2, s3):
    nb, h = idx_v.shape
    d = rows_v.shape[-1]
    scale = jnp.float32(math.sqrt(d))
    wid = lax.axis_index("s") * NUM_CORES + lax.axis_index("c")
    base = wid * nb
    gsem = (g0, g1, g2, g3)
    ssem = (s0, s1, s2, s3)

    def fire_gather(bi, slot):
        pltpu.async_copy(
            lut_hbm.at[idx_v.at[bi]], rows_v.at[slot], gsem[slot])

    def drain_gather(slot):
        pltpu.make_async_copy(
            lut_hbm.at[idx_v.at[0]], rows_v.at[slot], gsem[slot]).wait()

    def fire_store(bi, slot):
        pltpu.async_copy(
            rows_v.at[slot],
            out_hbm.at[base + bi, pl.ds(0, h), pl.ds(0, d)], ssem[slot])

    def drain_store(slot):
        pltpu.make_async_copy(
            rows_v.at[slot], out_hbm.at[0, pl.ds(0, h), pl.ds(0, d)],
            ssem[slot]).wait()

    def scale_slot(slot):
        def row_body(r, _):
            row = rows_v.at[slot, r]
            for k in range(d // LANES):
                sl = pl.ds(k * LANES, LANES)
                row[sl] = row[sl] * scale
            return 0

        lax.fori_loop(0, h, row_body, 0, unroll=2)

    pltpu.sync_copy(x_hbm.at[pl.ds(base, nb)], idx_v)

    fire_gather(0, 0)
    fire_gather(1, 1)

    def group_body(t, _):
        for p in range(NSLOT):
            bi = t * NSLOT + p
            q = (p + 2) % NSLOT
            if p < 2:
                @pl.when(t >= 1)
                def _():
                    drain_store(q)
                fire_gather(bi + 2, q)
            else:
                drain_store(q)

                @pl.when(t < (nb // NSLOT) - 1)
                def _():
                    fire_gather(bi + 2, q)
            drain_gather(p)
            scale_slot(p)
            fire_store(bi, p)
        return 0

    lax.fori_loop(0, nb // NSLOT, group_body, 0)

    drain_store(2)
    drain_store(3)


def kernel(x, lut):
    b, h = x.shape
    v, d = lut.shape
    nb = b // NUM_WORKERS
    assert b % (NUM_WORKERS * NSLOT) == 0
    assert d % LANES == 0
    hp = (h + 7) // 8 * 8
    dp = 128

    xi = x.astype(jnp.int32)

    mesh = plsc.VectorSubcoreMesh(core_axis_name="c", subcore_axis_name="s")
    run = pl.kernel(
        _emb_body,
        out_type=jax.ShapeDtypeStruct((b, hp, dp), jnp.float32),
        mesh=mesh,
        scratch_types=[
            pltpu.VMEM((nb, h), jnp.int32),
            pltpu.VMEM((NSLOT, h, d), jnp.float32),
        ] + [pltpu.SemaphoreType.DMA] * 8,
        compiler_params=pltpu.CompilerParams(use_tc_tiling_on_sc=False),
    )
    buf = run(xi, lut)
    return buf[:, :h, :d]


# speedup vs baseline: 2.4283x; 1.2150x over previous
import math

import jax
import jax.numpy as jnp
from jax import lax
from jax.experimental import layout as jax_layout
from jax.experimental import pallas as pl
from jax.experimental.pallas import tpu as pltpu
from jax.experimental.pallas import tpu_sc as plsc

NUM_CORES = 2
NUM_SUBCORES = 16
NUM_WORKERS = NUM_CORES * NUM_SUBCORES
LANES = 16
NSLOT = 4


def _emb_body(x_hbm, lut_hbm, out_hbm, idx_v, rows_v, g0, g1, g2, g3, s0,
              s1, s2, s3):
    nb, h = idx_v.shape
    d = rows_v.shape[-1]
    scale = jnp.float32(math.sqrt(d))
    wid = lax.axis_index("s") * NUM_CORES + lax.axis_index("c")
    base = wid * nb
    gsem = (g0, g1, g2, g3)
    ssem = (s0, s1, s2, s3)

    def fire_gather(bi, slot):
        pltpu.async_copy(
            lut_hbm.at[idx_v.at[bi]], rows_v.at[slot], gsem[slot])

    def drain_gather(slot):
        pltpu.make_async_copy(
            lut_hbm.at[idx_v.at[0]], rows_v.at[slot], gsem[slot]).wait()

    def fire_store(bi, slot):
        pltpu.async_copy(
            rows_v.at[slot],
            out_hbm.at[base + bi, pl.ds(0, h), pl.ds(0, d)], ssem[slot])

    def drain_store(slot):
        pltpu.make_async_copy(
            rows_v.at[slot], out_hbm.at[0, pl.ds(0, h), pl.ds(0, d)],
            ssem[slot]).wait()

    def scale_slot(slot):
        def row_body(r, _):
            row = rows_v.at[slot, r]
            for k in range(d // LANES):
                sl = pl.ds(k * LANES, LANES)
                row[sl] = row[sl] * scale
            return 0

        lax.fori_loop(0, h, row_body, 0, unroll=2)

    pltpu.sync_copy(x_hbm.at[pl.ds(base, nb)], idx_v)

    fire_gather(0, 0)
    fire_gather(1, 1)

    def group_body(t, _):
        for p in range(NSLOT):
            bi = t * NSLOT + p
            q = (p + 2) % NSLOT
            if p < 2:
                @pl.when(t >= 1)
                def _():
                    drain_store(q)
                fire_gather(bi + 2, q)
            else:
                drain_store(q)

                @pl.when(t < (nb // NSLOT) - 1)
                def _():
                    fire_gather(bi + 2, q)
            drain_gather(p)
            scale_slot(p)
            fire_store(bi, p)
        return 0

    lax.fori_loop(0, nb // NSLOT, group_body, 0)

    drain_store(2)
    drain_store(3)


def kernel(x, lut):
    b, h = x.shape
    v, d = lut.shape
    nb = b // NUM_WORKERS
    assert b % (NUM_WORKERS * NSLOT) == 0
    assert d % LANES == 0
    hp = (h + 7) // 8 * 8
    dp = 128

    xi = x.astype(jnp.int32)

    mesh = plsc.VectorSubcoreMesh(core_axis_name="c", subcore_axis_name="s")
    run = pl.kernel(
        _emb_body,
        out_type=jax.ShapeDtypeStruct((b, hp, dp), jnp.float32),
        mesh=mesh,
        scratch_types=[
            pltpu.VMEM((nb, h), jnp.int32),
            pltpu.VMEM((NSLOT, h, d), jnp.float32),
        ] + [pltpu.SemaphoreType.DMA] * 8,
        compiler_params=pltpu.CompilerParams(use_tc_tiling_on_sc=False),
    )
    buf = run(xi, lut)
    out = buf[:, :h, :d]
    return jax_layout.with_layout_constraint(
        out, jax_layout.Layout(major_to_minor=(0, 1, 2))
    )
